# Initial kernel scaffold; baseline (speedup 1.0000x reference)
import jax
import jax.numpy as jnp
from jax.experimental import pallas as pl


def kernel(user_idx, game_idx, user_emb_gmf, game_emb_gmf, user_emb_mlp, game_emb_mlp, W1, b1, W2, b2, Wf, bf):
    raise NotImplementedError("write your pallas kernel here")


# speedup vs baseline: 2.6586x; 2.6586x over previous
import functools

import jax
import jax.numpy as jnp
from jax import lax
from jax.experimental import pallas as pl
from jax.experimental.pallas import tpu as pltpu
from jax.experimental.pallas import tpu_sc as plsc

NUM_USERS = 100000
NUM_GAMES = 100000
EMB_DIM = 128
BATCH = 16384

NC = 2
NS = 16
NW = NC * NS
BPW = BATCH // NW
CHUNK = 128
NCHUNK = BPW // CHUNK


def _sc_gather(user_idx, game_idx, ug, vg, um, vm):
    uidx = user_idx.reshape(NW, NCHUNK, CHUNK)
    gidx = game_idx.reshape(NW, NCHUNK, CHUNK)

    mesh = plsc.VectorSubcoreMesh(
        core_axis_name="c", subcore_axis_name="s",
        num_cores=NC, num_subcores=NS)

    out_t = jax.ShapeDtypeStruct((BATCH, EMB_DIM), jnp.float32)

    @functools.partial(
        pl.kernel,
        out_type=(out_t, out_t, out_t, out_t),
        mesh=mesh,
        scratch_types=[
            pltpu.VMEM((NCHUNK, CHUNK), jnp.int32),
            pltpu.VMEM((NCHUNK, CHUNK), jnp.int32),
            pltpu.VMEM((BPW, EMB_DIM), jnp.float32),
            pltpu.SemaphoreType.DMA,
        ],
    )
    def body(uidx_hbm, gidx_hbm, ug_hbm, vg_hbm, um_hbm, vm_hbm,
             o_ug, o_vg, o_um, o_vm, idx_u, idx_g, buf, sem):
        wid = lax.axis_index("s") * NC + lax.axis_index("c")
        base = wid * BPW
        pltpu.sync_copy(uidx_hbm.at[wid], idx_u)
        pltpu.sync_copy(gidx_hbm.at[wid], idx_g)
        for tbl, idx, out in ((ug_hbm, idx_u, o_ug), (vg_hbm, idx_g, o_vg),
                              (um_hbm, idx_u, o_um), (vm_hbm, idx_g, o_vm)):
            handles = []
            for j in range(NCHUNK):
                handles.append(pltpu.async_copy(
                    tbl.at[idx.at[j]], buf.at[pl.ds(j * CHUNK, CHUNK)], sem))
            for h in handles:
                h.wait()
            pltpu.sync_copy(buf, out.at[pl.ds(base, BPW)])

    return body(uidx, gidx, ug, vg, um, vm)


BLK = 2048
NBLK = BATCH // BLK


def _tc_body(ug_ref, vg_ref, um_ref, vm_ref, w1t_ref, b1_ref, w2t_ref,
             b2_ref, wfg_ref, wfm_ref, bf_ref, out_ref):
    m1 = jnp.dot(um_ref[...], w1t_ref[:EMB_DIM, :],
                 preferred_element_type=jnp.float32)
    m1 = m1 + jnp.dot(vm_ref[...], w1t_ref[EMB_DIM:, :],
                      preferred_element_type=jnp.float32)
    m1 = jnp.maximum(m1 + b1_ref[...], 0.0)
    m2 = jnp.dot(m1, w2t_ref[...], preferred_element_type=jnp.float32)
    m2 = jnp.maximum(m2 + b2_ref[...], 0.0)
    g = ug_ref[...] * vg_ref[...]
    res = (jnp.sum(g * wfg_ref[...], axis=1)
           + jnp.sum(m2 * wfm_ref[...], axis=1) + bf_ref[0, 0])
    out_ref[...] = res.reshape(1, 1, BLK)


def _tc_forward(ug, vg, um, vm, W1, b1, W2, b2, Wf, bf):
    w1t = W1.T
    w2t = W2.T
    wfg = Wf[:, :EMB_DIM]
    wfm = Wf[:, EMB_DIM:]
    b1r = b1.reshape(1, -1)
    b2r = b2.reshape(1, -1)
    bfr = bf.reshape(1, 1)

    full = lambda shape: pl.BlockSpec(shape, lambda i: (0, 0))
    out = pl.pallas_call(
        _tc_body,
        grid=(NBLK,),
        in_specs=[
            pl.BlockSpec((BLK, EMB_DIM), lambda i: (i, 0)),
            pl.BlockSpec((BLK, EMB_DIM), lambda i: (i, 0)),
            pl.BlockSpec((BLK, EMB_DIM), lambda i: (i, 0)),
            pl.BlockSpec((BLK, EMB_DIM), lambda i: (i, 0)),
            full((2 * EMB_DIM, EMB_DIM)),
            full((1, 128)),
            full((EMB_DIM, 64)),
            full((1, 64)),
            full((1, EMB_DIM)),
            full((1, 64)),
            full((1, 1)),
        ],
        out_specs=pl.BlockSpec((1, 1, BLK), lambda i: (i, 0, 0)),
        out_shape=jax.ShapeDtypeStruct((NBLK, 1, BLK), jnp.float32),
    )(ug, vg, um, vm, w1t, b1r, w2t, b2r, wfg, wfm, bfr)
    return out.reshape(BATCH)


def kernel(user_idx, game_idx, user_emb_gmf, game_emb_gmf, user_emb_mlp,
           game_emb_mlp, W1, b1, W2, b2, Wf, bf):
    ug, vg, um, vm = _sc_gather(user_idx, game_idx, user_emb_gmf,
                                game_emb_gmf, user_emb_mlp, game_emb_mlp)
    return _tc_forward(ug, vg, um, vm, W1, b1, W2, b2, Wf, bf)
